# Initial kernel scaffold; baseline (speedup 1.0000x reference)
import jax
import jax.numpy as jnp
from jax.experimental import pallas as pl


def kernel(x, edge_index, Wl1, bl1, Wr1, g1, b1, Wl2, bl2, Wr2, g2, b2, Wl3, bl3, Wr3, g3, b3, Wl4, bl4, Wr4):
    raise NotImplementedError("write your pallas kernel here")


# speedup vs baseline: 7.2032x; 7.2032x over previous
import functools

import jax
import jax.numpy as jnp
from jax import lax
from jax.experimental import pallas as pl
from jax.experimental.pallas import tpu as pltpu
from jax.experimental.pallas import tpu_sc as plsc

N = 50000
E = 800000
H = 128
NP = 51200
BLK = 2048
CBN = 1.0 / (1.0 + 1e-5) ** 0.5

EWA = E // 32
ABLK = 2000

EW = E // 16
BLKE = 2000
CHUNK = 8192
NCHUNK = 8
NPB = CHUNK * NCHUNK
CR = CHUNK + 128
ZR = CR // 16
GB = 64
PCAP = EW + 4 * GB


def _ru64(x):
    return lax.shift_left(lax.shift_right_logical(x + (GB - 1), 6), 6)


SB = 128
NSB = EWA // SB
SBT = EWA - NSB * SB


def _seg_scalar(xflat, src, dst):
    mesh = plsc.VectorSubcoreMesh(core_axis_name="c", subcore_axis_name="s")

    @functools.partial(
        pl.kernel,
        out_type=jax.ShapeDtypeStruct((2, NP), jnp.float32),
        mesh=mesh,
        compiler_params=pltpu.CompilerParams(needs_layout_passes=False),
        scratch_types=[
            pltpu.VMEM((SB,), jnp.int32),
            pltpu.VMEM((SB,), jnp.int32),
            pltpu.VMEM((SB,), jnp.float32),
            pltpu.VMEM((SBT,), jnp.int32),
            pltpu.VMEM((SBT,), jnp.int32),
            pltpu.VMEM((SBT,), jnp.float32),
            pltpu.VMEM((SB,), jnp.float32),
            pltpu.VMEM_SHARED((NP,), jnp.float32),
            pltpu.SemaphoreType.DMA,
        ],
    )
    def k(x_hbm, src_hbm, dst_hbm, out_hbm,
          sidx, didx, vals, sidxt, didxt, valst, zbuf, acc, dsem):
        core = lax.axis_index("c")
        sub = lax.axis_index("s")
        wid = sub * 2 + core
        for j in range(SB // 16):
            zbuf[pl.ds(j * 16, 16)] = jnp.zeros((16,), jnp.float32)
        for j in range(NP // 16 // SB):
            pltpu.sync_copy(
                zbuf, acc.at[pl.ds(sub * (NP // 16) + j * SB, SB)])
        plsc.subcore_barrier()

        ebase = wid * EWA

        def bat(b, _):
            off = ebase + b * SB
            pltpu.sync_copy(src_hbm.at[pl.ds(off, SB)], sidx)
            pltpu.sync_copy(dst_hbm.at[pl.ds(off, SB)], didx)
            pltpu.async_copy(x_hbm.at[sidx], vals, dsem).wait()
            pltpu.sync_copy(vals, acc.at[didx], add=True)
            return 0

        lax.fori_loop(0, NSB, bat, 0)
        off = ebase + NSB * SB
        pltpu.sync_copy(src_hbm.at[pl.ds(off, SBT)], sidxt)
        pltpu.sync_copy(dst_hbm.at[pl.ds(off, SBT)], didxt)
        pltpu.async_copy(x_hbm.at[sidxt], valst, dsem).wait()
        pltpu.sync_copy(valst, acc.at[didxt], add=True)

        plsc.subcore_barrier()
        pltpu.sync_copy(acc.at[pl.ds(sub * (NP // 16), NP // 16)],
                        out_hbm.at[core, pl.ds(sub * (NP // 16), NP // 16)])

    return k(xflat, src, dst)


def _seg128(h, src, dst, zrows):
    mesh = plsc.VectorSubcoreMesh(core_axis_name="c", subcore_axis_name="s")

    @functools.partial(
        pl.kernel,
        out_type=jax.ShapeDtypeStruct((NPB, H), jnp.float32),
        mesh=mesh,
        compiler_params=pltpu.CompilerParams(needs_layout_passes=False),
        scratch_types=[
            pltpu.VMEM((PCAP,), jnp.int32),
            pltpu.VMEM((BLKE,), jnp.int32),
            pltpu.VMEM((BLKE,), jnp.int32),
            pltpu.VMEM((GB, H), jnp.float32),
            pltpu.VMEM((GB,), jnp.int32),
            pltpu.VMEM((GB,), jnp.int32),
            pltpu.VMEM_SHARED((CR, H), jnp.float32),
            pltpu.SemaphoreType.DMA,
        ],
    )
    def k(h_hbm, src_hbm, dst_hbm, z_hbm, out_hbm,
          pend, sblk, dblk, rows, sidx, lidx, chunk, dsem):
        core = lax.axis_index("c")
        sub = lax.axis_index("s")
        ebase = sub * EW
        zero16 = jnp.zeros((16,), jnp.int32)

        def blk1(b, carry):
            pltpu.sync_copy(dst_hbm.at[pl.ds(ebase + b * BLKE, BLKE)], dblk)

            def it(i, cy):
                d16 = dblk[pl.ds(i * 16, 16)]
                cid = lax.shift_right_logical(d16, 13)
                out = []
                for p in range(4):
                    m = cid == (2 * p + core)
                    out.append(cy[p] + plsc.all_reduce_population_count(m))
                return tuple(out)

            return lax.fori_loop(0, BLKE // 16, it, carry)

        cvecs = lax.fori_loop(0, EW // BLKE, blk1,
                              (zero16, zero16, zero16, zero16))
        cnts = [jnp.max(cv) for cv in cvecs]
        offs = []
        o = jnp.int32(0)
        for p in range(4):
            offs.append(o)
            o = o + _ru64(cnts[p])

        def blk2(b, carry):
            boff = ebase + b * BLKE
            pltpu.sync_copy(src_hbm.at[pl.ds(boff, BLKE)], sblk)
            pltpu.sync_copy(dst_hbm.at[pl.ds(boff, BLKE)], dblk)

            def it(i, cy):
                s16 = sblk[pl.ds(i * 16, 16)]
                d16 = dblk[pl.ds(i * 16, 16)]
                cid = lax.shift_right_logical(d16, 13)
                out = []
                for p in range(4):
                    tgt = 2 * p + core
                    m = cid == tgt
                    packed = lax.shift_left(d16 - tgt * CHUNK, 16) | s16
                    plsc.store_compressed(pend.at[pl.ds(cy[p], 16)], packed,
                                          mask=m)
                    out.append(
                        cy[p] + jnp.max(plsc.all_reduce_population_count(m)))
                return tuple(out)

            return lax.fori_loop(0, BLKE // 16, it, carry)

        ends = lax.fori_loop(0, EW // BLKE, blk2, tuple(offs))

        iota16 = jnp.arange(16, dtype=jnp.int32)
        safe = jnp.full((16,), CHUNK << 16, dtype=jnp.int32)
        for p in range(4):
            padn = _ru64(cnts[p]) - cnts[p]
            for j in range(4):
                m = (j * 16 + iota16) < padn
                plsc.store_compressed(pend.at[pl.ds(ends[p] + j * 16, 16)],
                                      safe, mask=m)

        for p in range(4):
            cid = 2 * p + core
            pltpu.sync_copy(z_hbm, chunk.at[pl.ds(sub * ZR, ZR)])
            plsc.subcore_barrier()

            def bat(bi, _, offp=offs[p]):
                pbase = offp + bi * GB
                for j in range(GB // 16):
                    pk = pend[pl.ds(pbase + j * 16, 16)]
                    sidx[pl.ds(j * 16, 16)] = pk & 0xFFFF
                    lidx[pl.ds(j * 16, 16)] = lax.shift_right_logical(pk, 16)
                pltpu.async_copy(h_hbm.at[sidx], rows, dsem).wait()
                pltpu.sync_copy(rows, chunk.at[lidx], add=True)
                return 0

            nb = lax.shift_right_logical(_ru64(cnts[p]), 6)
            lax.fori_loop(0, nb, bat, 0)
            plsc.subcore_barrier()
            pltpu.sync_copy(
                chunk.at[pl.ds(sub * (CHUNK // 16), CHUNK // 16)],
                out_hbm.at[pl.ds(cid * CHUNK + sub * (CHUNK // 16),
                                 CHUNK // 16)])
            plsc.subcore_barrier()

    return k(h, src, dst, zrows)


def _tc1_body(part_ref, xc_ref, u_ref, v_ref, bl_ref, g_ref, b_ref, o_ref):
    ones = jnp.ones((2, 1), jnp.float32)
    s_col = lax.dot_general(part_ref[...], ones, (((0,), (0,)), ((), ())),
                            preferred_element_type=jnp.float32)
    pre = s_col * u_ref[...] + xc_ref[...] * v_ref[...] + bl_ref[...]
    o_ref[...] = jnp.maximum(pre * (g_ref[...] * CBN) + b_ref[...], 0.0)


def _tc1(part, xc, u, v, bl, g, b):
    grid = (NP // BLK,)
    return pl.pallas_call(
        _tc1_body,
        grid=grid,
        in_specs=[
            pl.BlockSpec((2, BLK), lambda i: (0, i)),
            pl.BlockSpec((BLK, 1), lambda i: (i, 0)),
            pl.BlockSpec((1, H), lambda i: (0, 0)),
            pl.BlockSpec((1, H), lambda i: (0, 0)),
            pl.BlockSpec((1, H), lambda i: (0, 0)),
            pl.BlockSpec((1, H), lambda i: (0, 0)),
            pl.BlockSpec((1, H), lambda i: (0, 0)),
        ],
        out_specs=pl.BlockSpec((BLK, H), lambda i: (i, 0)),
        out_shape=jax.ShapeDtypeStruct((NP, H), jnp.float32),
    )(part, xc, u, v, bl, g, b)


def _tc_mid_body(agg_ref, h_ref, wl_ref, wr_ref, bl_ref, g_ref, b_ref, o_ref):
    pre = (jnp.dot(agg_ref[...], wl_ref[...],
                   preferred_element_type=jnp.float32)
           + jnp.dot(h_ref[...], wr_ref[...],
                     preferred_element_type=jnp.float32)
           + bl_ref[...])
    o_ref[...] = jnp.maximum(pre * (g_ref[...] * CBN) + b_ref[...], 0.0)


def _tc_mid(agg, h, wlT, wrT, bl, g, b):
    grid = (NP // BLK,)
    return pl.pallas_call(
        _tc_mid_body,
        grid=grid,
        in_specs=[
            pl.BlockSpec((BLK, H), lambda i: (i, 0)),
            pl.BlockSpec((BLK, H), lambda i: (i, 0)),
            pl.BlockSpec((H, H), lambda i: (0, 0)),
            pl.BlockSpec((H, H), lambda i: (0, 0)),
            pl.BlockSpec((1, H), lambda i: (0, 0)),
            pl.BlockSpec((1, H), lambda i: (0, 0)),
            pl.BlockSpec((1, H), lambda i: (0, 0)),
        ],
        out_specs=pl.BlockSpec((BLK, H), lambda i: (i, 0)),
        out_shape=jax.ShapeDtypeStruct((NP, H), jnp.float32),
    )(agg, h, wlT, wrT, bl, g, b)


def _tc3_body(agg_ref, h_ref, wl_ref, wr_ref, bl_ref, g_ref, b_ref,
              w4l_ref, w4r_ref, p_ref, r_ref):
    pre = (jnp.dot(agg_ref[...], wl_ref[...],
                   preferred_element_type=jnp.float32)
           + jnp.dot(h_ref[...], wr_ref[...],
                     preferred_element_type=jnp.float32)
           + bl_ref[...])
    h3 = jnp.maximum(pre * (g_ref[...] * CBN) + b_ref[...], 0.0)
    p_ref[...] = jnp.sum(h3 * w4l_ref[...], axis=1, keepdims=True)
    r_ref[...] = jnp.sum(h3 * w4r_ref[...], axis=1, keepdims=True)


def _tc3(agg, h, wlT, wrT, bl, g, b, w4l, w4r):
    grid = (NP // BLK,)
    return pl.pallas_call(
        _tc3_body,
        grid=grid,
        in_specs=[
            pl.BlockSpec((BLK, H), lambda i: (i, 0)),
            pl.BlockSpec((BLK, H), lambda i: (i, 0)),
            pl.BlockSpec((H, H), lambda i: (0, 0)),
            pl.BlockSpec((H, H), lambda i: (0, 0)),
            pl.BlockSpec((1, H), lambda i: (0, 0)),
            pl.BlockSpec((1, H), lambda i: (0, 0)),
            pl.BlockSpec((1, H), lambda i: (0, 0)),
            pl.BlockSpec((1, H), lambda i: (0, 0)),
            pl.BlockSpec((1, H), lambda i: (0, 0)),
        ],
        out_specs=[
            pl.BlockSpec((BLK, 1), lambda i: (i, 0)),
            pl.BlockSpec((BLK, 1), lambda i: (i, 0)),
        ],
        out_shape=[
            jax.ShapeDtypeStruct((NP, 1), jnp.float32),
            jax.ShapeDtypeStruct((NP, 1), jnp.float32),
        ],
    )(agg, h, wlT, wrT, bl, g, b, w4l, w4r)


def _tc4_body(part_ref, r_ref, bl_ref, o_ref):
    ones = jnp.ones((2, 1), jnp.float32)
    t_col = lax.dot_general(part_ref[...], ones, (((0,), (0,)), ((), ())),
                            preferred_element_type=jnp.float32)
    o_ref[...] = t_col + r_ref[...] + bl_ref[...]


def _tc4(part, r, bl):
    grid = (NP // BLK,)
    return pl.pallas_call(
        _tc4_body,
        grid=grid,
        in_specs=[
            pl.BlockSpec((2, BLK), lambda i: (0, i)),
            pl.BlockSpec((BLK, 1), lambda i: (i, 0)),
            pl.BlockSpec((1, 1), lambda i: (0, 0)),
        ],
        out_specs=pl.BlockSpec((BLK, 1), lambda i: (i, 0)),
        out_shape=jax.ShapeDtypeStruct((NP, 1), jnp.float32),
    )(part, r, bl)


def kernel(x, edge_index, Wl1, bl1, Wr1, g1, b1, Wl2, bl2, Wr2, g2, b2,
           Wl3, bl3, Wr3, g3, b3, Wl4, bl4, Wr4):
    src = edge_index[0]
    dst = edge_index[1]
    xc = jnp.pad(x, ((0, NP - N), (0, 0)))
    zrows = jnp.zeros((ZR, H), jnp.float32)

    part1 = _seg_scalar(xc[:, 0], src, dst)
    h1 = _tc1(part1, xc, Wl1.T, Wr1.T, bl1[None, :], g1[None, :], b1[None, :])

    agg2 = _seg128(h1, src, dst, zrows)[:NP]
    h2 = _tc_mid(agg2, h1, Wl2.T, Wr2.T, bl2[None, :], g2[None, :],
                 b2[None, :])

    agg3 = _seg128(h2, src, dst, zrows)[:NP]
    p3, r3 = _tc3(agg3, h2, Wl3.T, Wr3.T, bl3[None, :], g3[None, :],
                  b3[None, :], Wl4, Wr4)

    part4 = _seg_scalar(p3[:, 0], src, dst)
    out = _tc4(part4, r3, bl4[None, :])
    return out[:N]
